# Initial kernel scaffold; baseline (speedup 1.0000x reference)
import jax
import jax.numpy as jnp
from jax.experimental import pallas as pl


def kernel(intent_ids, table):
    raise NotImplementedError("write your pallas kernel here")


# speedup vs baseline: 4.9945x; 4.9945x over previous
import functools

import jax
import jax.numpy as jnp
from jax import lax
from jax.experimental import pallas as pl
from jax.experimental.pallas import tpu as pltpu
from jax.experimental.pallas import tpu_sc as plsc

NUM_INTENTS = 100000
EMBED_DIM = 64
BATCH = 16384
SEQ_LEN = 200

LANE = 128
TOTAL = BATCH * SEQ_LEN
NROWS = TOTAL // LANE
NW = 32
ROWS_PER_W = NROWS // NW
K = 8
GROUPS = ROWS_PER_W // K


def _gather_body(table_hbm, idx_hbm, out_hbm, idx_v, rows_v, sem):
    wid = lax.axis_index("s") * 2 + lax.axis_index("c")
    base = wid * ROWS_PER_W

    def body(g, carry):
        row = base + g * K
        pltpu.sync_copy(idx_hbm.at[pl.ds(row, K)], idx_v)
        cps = [
            pltpu.async_copy(table_hbm.at[idx_v.at[j]], rows_v.at[j], sem)
            for j in range(K)
        ]
        for cp in cps:
            cp.wait()
        pltpu.sync_copy(rows_v, out_hbm.at[pl.ds(row, K)])
        return carry

    lax.fori_loop(0, GROUPS, body, 0)


@jax.jit
def _gather(table, idx2d):
    mesh = plsc.VectorSubcoreMesh(core_axis_name="c", subcore_axis_name="s")
    return pl.kernel(
        _gather_body,
        mesh=mesh,
        out_type=jax.ShapeDtypeStruct((NROWS, LANE, EMBED_DIM), jnp.float32),
        scratch_types=[
            pltpu.VMEM((K, LANE), jnp.int32),
            pltpu.VMEM((K, LANE, EMBED_DIM), jnp.float32),
            pltpu.SemaphoreType.DMA,
        ],
        compiler_params=pltpu.CompilerParams(use_tc_tiling_on_sc=False),
    )(table, idx2d)


def kernel(intent_ids, table):
    idx2d = intent_ids.reshape(NROWS, LANE)
    out = _gather(table, idx2d)
    return out.reshape(BATCH, SEQ_LEN, EMBED_DIM)


# speedup vs baseline: 5.0835x; 1.0178x over previous
import jax
import jax.numpy as jnp
from jax import lax
from jax.experimental import pallas as pl
from jax.experimental.pallas import tpu as pltpu
from jax.experimental.pallas import tpu_sc as plsc

NUM_INTENTS = 100000
EMBED_DIM = 64
BATCH = 16384
SEQ_LEN = 200

LANE = 128
TOTAL = BATCH * SEQ_LEN
NROWS = TOTAL // LANE
NW = 32
ROWS_PER_W = NROWS // NW
K = 5
GROUPS = ROWS_PER_W // K


def _gather_body(table_hbm, idx_hbm, out_hbm, idx_v, rows_v,
                 gsem0, gsem1, ssem0, ssem1):
    wid = lax.axis_index("s") * 2 + lax.axis_index("c")
    base = wid * ROWS_PER_W
    gsems = (gsem0, gsem1)
    ssems = (ssem0, ssem1)

    def load_idx(g, b):
        pltpu.sync_copy(idx_hbm.at[pl.ds(base + g * K, K)], idx_v.at[b])

    def fire_gathers(b):
        for j in range(K):
            pltpu.async_copy(table_hbm.at[idx_v.at[b].at[j]],
                             rows_v.at[b].at[j], gsems[b])

    def wait_gathers(b):
        for j in range(K):
            pltpu.make_async_copy(table_hbm.at[pl.ds(0, LANE)],
                                  rows_v.at[b].at[j], gsems[b]).wait()

    def store(g, b):
        pltpu.async_copy(rows_v.at[b], out_hbm.at[pl.ds(base + g * K, K)],
                         ssems[b])

    def wait_store(b):
        pltpu.make_async_copy(rows_v.at[b], out_hbm.at[pl.ds(0, K)],
                              ssems[b]).wait()

    load_idx(0, 0)
    fire_gathers(0)
    load_idx(1, 1)
    fire_gathers(1)
    wait_gathers(0)
    store(0, 0)

    def loop_body(t, carry):
        for b in range(2):
            g = 2 * t + 2 + b
            b2 = 1 - b
            wait_store(b)
            load_idx(g, b)
            fire_gathers(b)
            wait_gathers(b2)
            store(g - 1, b2)
        return carry

    lax.fori_loop(0, (GROUPS - 2) // 2, loop_body, 0)

    last_b = (GROUPS - 1) % 2
    wait_gathers(last_b)
    store(GROUPS - 1, last_b)
    wait_store(1 - last_b)
    wait_store(last_b)


@jax.jit
def _gather(table, idx2d):
    mesh = plsc.VectorSubcoreMesh(core_axis_name="c", subcore_axis_name="s")
    return pl.kernel(
        _gather_body,
        mesh=mesh,
        out_type=jax.ShapeDtypeStruct((NROWS, LANE, EMBED_DIM), jnp.float32),
        scratch_types=[
            pltpu.VMEM((2, K, LANE), jnp.int32),
            pltpu.VMEM((2, K, LANE, EMBED_DIM), jnp.float32),
            pltpu.SemaphoreType.DMA,
            pltpu.SemaphoreType.DMA,
            pltpu.SemaphoreType.DMA,
            pltpu.SemaphoreType.DMA,
        ],
        compiler_params=pltpu.CompilerParams(use_tc_tiling_on_sc=False),
    )(table, idx2d)


def kernel(intent_ids, table):
    idx2d = intent_ids.reshape(NROWS, LANE)
    out = _gather(table, idx2d)
    return out.reshape(BATCH, SEQ_LEN, EMBED_DIM)


# speedup vs baseline: 5.0889x; 1.0011x over previous
import jax
import jax.numpy as jnp
from jax import lax
from jax.experimental import pallas as pl
from jax.experimental.pallas import tpu as pltpu
from jax.experimental.pallas import tpu_sc as plsc

NUM_INTENTS = 100000
EMBED_DIM = 64
BATCH = 16384
SEQ_LEN = 200

LANE = 128
TOTAL = BATCH * SEQ_LEN
NROWS = TOTAL // LANE
NW = 32
ROWS_PER_W = NROWS // NW
K = 5
GROUPS = ROWS_PER_W // K


def _gather_body(table_hbm, idx_hbm, out_hbm, idx_v, rows_v,
                 gsem0, gsem1, ssem0, ssem1):
    wid = lax.axis_index("s") * 2 + lax.axis_index("c")
    base = wid * ROWS_PER_W
    gsems = (gsem0, gsem1)
    ssems = (ssem0, ssem1)

    def load_idx(g, b):
        pltpu.sync_copy(idx_hbm.at[pl.ds(base + g * K, K)], idx_v.at[b])

    def fire_gathers(b):
        for j in range(K):
            pltpu.async_copy(table_hbm.at[idx_v.at[b].at[j]],
                             rows_v.at[b].at[pl.ds(j * LANE, LANE)], gsems[b])

    def wait_gathers(b):
        for j in range(K):
            pltpu.make_async_copy(table_hbm.at[pl.ds(0, LANE)],
                                  rows_v.at[b].at[pl.ds(j * LANE, LANE)],
                                  gsems[b]).wait()

    def store(g, b):
        pltpu.async_copy(rows_v.at[b],
                         out_hbm.at[pl.ds((base + g * K) * LANE, K * LANE)],
                         ssems[b])

    def wait_store(b):
        pltpu.make_async_copy(rows_v.at[b], out_hbm.at[pl.ds(0, K * LANE)],
                              ssems[b]).wait()

    load_idx(0, 0)
    fire_gathers(0)
    load_idx(1, 1)
    fire_gathers(1)
    wait_gathers(0)
    store(0, 0)

    def loop_body(t, carry):
        for b in range(2):
            g = 2 * t + 2 + b
            b2 = 1 - b
            wait_store(b)
            load_idx(g, b)
            fire_gathers(b)
            wait_gathers(b2)
            store(g - 1, b2)
        return carry

    lax.fori_loop(0, (GROUPS - 2) // 2, loop_body, 0)

    last_b = (GROUPS - 1) % 2
    wait_gathers(last_b)
    store(GROUPS - 1, last_b)
    wait_store(1 - last_b)
    wait_store(last_b)


@jax.jit
def _gather(table, idx2d):
    mesh = plsc.VectorSubcoreMesh(core_axis_name="c", subcore_axis_name="s")
    return pl.kernel(
        _gather_body,
        mesh=mesh,
        out_type=jax.ShapeDtypeStruct((TOTAL, EMBED_DIM), jnp.float32),
        scratch_types=[
            pltpu.VMEM((2, K, LANE), jnp.int32),
            pltpu.VMEM((2, K * LANE, EMBED_DIM), jnp.float32),
            pltpu.SemaphoreType.DMA,
            pltpu.SemaphoreType.DMA,
            pltpu.SemaphoreType.DMA,
            pltpu.SemaphoreType.DMA,
        ],
        compiler_params=pltpu.CompilerParams(use_tc_tiling_on_sc=False),
    )(table, idx2d)


def kernel(intent_ids, table):
    idx2d = intent_ids.reshape(NROWS, LANE)
    out = _gather(table, idx2d)
    return out.reshape(BATCH, SEQ_LEN, EMBED_DIM)
